# baseline (speedup 1.0000x reference)
import functools
import jax
import jax.numpy as jnp
from jax import lax
from jax.experimental import pallas as pl
from jax.experimental.pallas import tpu as pltpu
from jax.experimental.pallas import tpu_sc as plsc

DIM_W = 100000
HID = 32
BATCH = 16384
NCAT = 26
B_TOT = BATCH * NCAT
NC = 2
NS = 16
NW = NC * NS
CH = 4096
N_CH = B_TOT // CH
CPC = BATCH // CH

_mesh = plsc.VectorSubcoreMesh(core_axis_name="c", subcore_axis_name="s")


@functools.partial(
    pl.kernel,
    mesh=_mesh,
    out_type=jax.ShapeDtypeStruct((NCAT, HID, BATCH), jnp.float32),
    scratch_types=[
        pltpu.VMEM((DIM_W,), jnp.float32),
        pltpu.VMEM((2, CH), jnp.int32),
        pltpu.VMEM((2, CH), jnp.float32),
        pltpu.SemaphoreType.DMA,
        pltpu.SemaphoreType.DMA,
    ],
    compiler_params=pltpu.CompilerParams(
        use_tc_tiling_on_sc=True, needs_layout_passes=False
    ),
)
def _sc_colgather(idx_hbm, wt_hbm, out_hbm, col_v, idx_v, out_v, isem, osem):
    h = lax.axis_index("s") * NC + lax.axis_index("c")
    pltpu.sync_copy(wt_hbm.at[h], col_v)

    def fire_idx(t):
        return pltpu.async_copy(
            idx_hbm.at[pl.ds(t * CH, CH)], idx_v.at[t % 2], isem
        )

    def fire_out(t):
        c = t // CPC
        b0 = (t % CPC) * CH
        return pltpu.async_copy(
            out_v.at[t % 2], out_hbm.at[c, h, pl.ds(b0, CH)], osem
        )

    def gather_chunk(t):
        buf = t % 2

        def body(v, carry):
            ii = idx_v[buf, pl.ds(v * 16, 16)]
            out_v[buf, pl.ds(v * 16, 16)] = plsc.load_gather(col_v, [ii])
            return carry

        lax.fori_loop(0, CH // 16, body, 0, unroll=8)

    pend_i = fire_idx(0)
    pend_o = None
    for t in range(N_CH):
        pend_i.wait()
        if t + 1 < N_CH:
            pend_i = fire_idx(t + 1)
        if pend_o is not None:
            pend_o.wait()
        gather_chunk(t)
        new_o = fire_out(t)
        if pend_o is None:
            pend_o = new_o
        else:
            pend_o = new_o
    pend_o.wait()


def kernel(x, W):
    idx_cb = x[:, 0, :].T.reshape(B_TOT)
    out = _sc_colgather(idx_cb, W.T)
    return out.transpose(2, 0, 1)


# speedup vs baseline: 2.3875x; 2.3875x over previous
import functools
import jax
import jax.numpy as jnp
from jax import lax
from jax.experimental import pallas as pl
from jax.experimental.pallas import tpu as pltpu
from jax.experimental.pallas import tpu_sc as plsc

DIM_W = 100000
HID = 32
BATCH = 16384
NCAT = 26
B_TOT = BATCH * NCAT
NC = 2
NS = 16
NW = NC * NS
CH = 4096
GRP = 16
N_CH = B_TOT // CH
CPC = BATCH // CH

_mesh = plsc.VectorSubcoreMesh(core_axis_name="c", subcore_axis_name="s")


@functools.partial(
    pl.kernel,
    mesh=_mesh,
    out_type=jax.ShapeDtypeStruct((NCAT, HID, BATCH), jnp.float32),
    scratch_types=[
        pltpu.VMEM((DIM_W,), jnp.float32),
        pltpu.VMEM((2, CH), jnp.int32),
        pltpu.VMEM((2, CH), jnp.float32),
        pltpu.SemaphoreType.DMA,
        pltpu.SemaphoreType.DMA,
    ],
    compiler_params=pltpu.CompilerParams(
        use_tc_tiling_on_sc=True, needs_layout_passes=False
    ),
)
def _sc_colgather(idx_hbm, wt_hbm, out_hbm, col_v, idx_v, out_v, isem, osem):
    h = lax.axis_index("s") * NC + lax.axis_index("c")
    pltpu.sync_copy(wt_hbm.at[h], col_v)

    def fire_idx(t, slot):
        @pl.when(t < N_CH)
        def _():
            pltpu.async_copy(
                idx_hbm.at[pl.ds(t * CH, CH)], idx_v.at[slot], isem
            )

    def wait_idx(slot):
        pltpu.make_async_copy(
            idx_hbm.at[pl.ds(0, CH)], idx_v.at[slot], isem
        ).wait()

    def fire_out(c, b, slot):
        pltpu.async_copy(
            out_v.at[slot], out_hbm.at[c, h, pl.ds(b * CH, CH)], osem
        )

    def wait_out(slot):
        pltpu.make_async_copy(
            out_v.at[slot], out_hbm.at[0, 0, pl.ds(0, CH)], osem
        ).wait()

    def gather_chunk(slot):
        def body(v, carry):
            base = v * (16 * GRP)
            iis = [idx_v[slot, pl.ds(base + u * 16, 16)] for u in range(GRP)]
            vals = [plsc.load_gather(col_v, [ii]) for ii in iis]
            for u in range(GRP):
                out_v[slot, pl.ds(base + u * 16, 16)] = vals[u]
            return carry

        lax.fori_loop(0, CH // (16 * GRP), body, 0)

    fire_idx(0, 0)
    fire_idx(1, 1)
    for b in range(CPC):
        s = b % 2
        wait_idx(s)
        if b >= 2:
            wait_out(s)
        gather_chunk(s)
        fire_out(0, b, s)
        fire_idx(b + 2, s)

    def c_body(c, carry):
        t0 = c * CPC
        for b in range(CPC):
            s = b % 2
            wait_idx(s)
            wait_out(s)
            gather_chunk(s)
            fire_out(c, b, s)
            fire_idx(t0 + b + 2, s)
        return carry

    lax.fori_loop(1, NCAT, c_body, 0)
    wait_out(0)
    wait_out(1)


def kernel(x, W):
    idx_cb = x[:, 0, :].T.reshape(B_TOT)
    out = _sc_colgather(idx_cb, W.T)
    return out.transpose(2, 0, 1)
